# Initial kernel scaffold; baseline (speedup 1.0000x reference)
import jax
import jax.numpy as jnp
from jax.experimental import pallas as pl


def kernel(draft_probs, target_probs, draft_token_ids, bonus_token_ids, num_draft_tokens, uniform_samples, residual_uniform):
    raise NotImplementedError("write your pallas kernel here")


# speedup vs baseline: 1.2933x; 1.2933x over previous
import jax
import jax.numpy as jnp
from jax import lax
from jax.experimental import pallas as pl
from jax.experimental.pallas import tpu as pltpu
from jax.experimental.pallas import tpu_sc as plsc

B = 128
V = 100000
CH = 20000
NCH = V // CH
SUB = 400
GPS = SUB // 16
SPC = CH // SUB
SPR = V // SUB
SPAD = 256

_info = plsc.get_sparse_core_info()
NC, NS = _info.num_cores, _info.num_subcores
NW = NC * NS
RPW = B // NW

_BIG = 3.0e38


def _iota16():
    return lax.iota(jnp.int32, 16)


def _get1(ref, idx):
    return plsc.load_gather(ref, [jnp.full((16,), idx, jnp.int32)])[0]


def _set1(ref, idx, val):
    plsc.store_scatter(ref, [jnp.full((16,), idx, jnp.int32)],
                       jnp.full((16,), val), mask=_iota16() == 0)


def _sc_body(d_hbm, t_hbm, ids_hbm, bon_hbm, u_hbm, ru_hbm, out_hbm,
             dbuf, tbuf, d3, t3, dq, tq, ids_v, bon_v, u_v, ru_v,
             subsums, outbuf):
    wid = lax.axis_index("s") * NC + lax.axis_index("c")

    pltpu.sync_copy(ids_hbm, ids_v)
    pltpu.sync_copy(bon_hbm, bon_v)
    pltpu.sync_copy(u_hbm, u_v)
    pltpu.sync_copy(ru_hbm, ru_v)

    outacc = jnp.zeros((16,), jnp.int32)
    for i in range(RPW):
        r = wid * RPW + i

        tid = _get1(ids_v, r)
        base8 = jnp.minimum((tid // 8) * 8, V - 16)
        pltpu.sync_copy(d_hbm.at[r, pl.ds(base8, 16)], dq)
        pltpu.sync_copy(t_hbm.at[r, pl.ds(base8, 16)], tq)
        off = tid - base8
        q = _get1(dq, off)
        p = _get1(tq, off)
        accept = (_get1(u_v, r) * q) < p

        def chunk_body(c, running):
            pltpu.sync_copy(d_hbm.at[r, pl.ds(c * CH, CH)], dbuf)
            pltpu.sync_copy(t_hbm.at[r, pl.ds(c * CH, CH)], tbuf)

            def sub_body(s, run2):
                def grp(g, acc):
                    o = s * SUB + g * 16
                    dd = dbuf[pl.ds(o, 16)]
                    tt = tbuf[pl.ds(o, 16)]
                    return acc + jnp.maximum(tt - dd, jnp.float32(0.0))

                acc = lax.fori_loop(0, GPS, grp,
                                    jnp.zeros((16,), jnp.float32))
                run2 = run2 + jnp.sum(acc)
                _set1(subsums, c * SPC + s, run2)
                return run2

            return lax.fori_loop(0, SPC, sub_body, running)

        total = lax.fori_loop(0, NCH, chunk_body, jnp.float32(0.0))
        plsc.store_scatter(subsums, [_iota16() + SPR],
                           jnp.full((16,), jnp.float32(_BIG)),
                           mask=_iota16() < (SPAD - SPR))
        thresh = _get1(ru_v, r) * total

        def p2(j, cnt):
            v = subsums[pl.ds(j * 16, 16)]
            return cnt + plsc.all_reduce_population_count(v < thresh)

        cntv = lax.fori_loop(0, SPAD // 16, p2,
                             jnp.zeros((16,), jnp.int32))
        gidx = jnp.minimum(jnp.max(cntv), SPR - 1)
        base = jnp.where(gidx > 0,
                         _get1(subsums, jnp.maximum(gidx - 1, 0)),
                         jnp.float32(0.0))

        pltpu.sync_copy(d_hbm.at[r, pl.ds(gidx * SUB, SUB)], d3)
        pltpu.sync_copy(t_hbm.at[r, pl.ds(gidx * SUB, SUB)], t3)

        def p3(g, carry):
            b, cv = carry
            dd = d3[pl.ds(g * 16, 16)]
            tt = t3[pl.ds(g * 16, 16)]
            res = jnp.maximum(tt - dd, jnp.float32(0.0))
            cum = plsc.cumsum(res)
            cv = cv + plsc.all_reduce_population_count((b + cum) < thresh)
            return b + jnp.sum(res), cv

        _, cntv3 = lax.fori_loop(0, GPS, p3,
                                 (base, jnp.zeros((16,), jnp.int32)))
        recovered = jnp.minimum(gidx * SUB + jnp.max(cntv3), V - 1)

        out0 = jnp.where(accept, tid, recovered)
        out1 = jnp.where(accept, _get1(bon_v, r), jnp.int32(-1))
        outacc = jnp.where(_iota16() == 2 * i, jnp.full((16,), out0), outacc)
        outacc = jnp.where(_iota16() == 2 * i + 1, jnp.full((16,), out1),
                           outacc)

    outbuf[...] = outacc
    pltpu.sync_copy(outbuf.at[pl.ds(0, 2 * RPW)],
                    out_hbm.at[pl.ds(wid * 2 * RPW, 2 * RPW)])


@jax.jit
def _sc_sampler(draft_probs, target_probs, draft_token_ids,
                bonus_token_ids, uniform_samples, residual_uniform):
    mesh = plsc.VectorSubcoreMesh(core_axis_name="c", subcore_axis_name="s")
    return pl.kernel(
        _sc_body,
        out_type=jax.ShapeDtypeStruct((B * 2,), jnp.int32),
        mesh=mesh,
        compiler_params=pltpu.CompilerParams(use_tc_tiling_on_sc=False,
                                             needs_layout_passes=False),
        scratch_types=[
            pltpu.VMEM((CH,), jnp.float32),
            pltpu.VMEM((CH,), jnp.float32),
            pltpu.VMEM((SUB,), jnp.float32),
            pltpu.VMEM((SUB,), jnp.float32),
            pltpu.VMEM((16,), jnp.float32),
            pltpu.VMEM((16,), jnp.float32),
            pltpu.VMEM((B,), jnp.int32),
            pltpu.VMEM((B,), jnp.int32),
            pltpu.VMEM((B,), jnp.float32),
            pltpu.VMEM((B,), jnp.float32),
            pltpu.VMEM((SPAD,), jnp.float32),
            pltpu.VMEM((16,), jnp.int32),
        ],
    )(draft_probs, target_probs, draft_token_ids, bonus_token_ids,
      uniform_samples, residual_uniform)


def kernel(draft_probs, target_probs, draft_token_ids, bonus_token_ids,
           num_draft_tokens, uniform_samples, residual_uniform):
    del num_draft_tokens
    flat = _sc_sampler(draft_probs, target_probs, draft_token_ids,
                       bonus_token_ids, uniform_samples, residual_uniform)
    output_token_ids = flat.reshape(B, 2)
    accept = (output_token_ids[:, 1] != jnp.int32(-1)).astype(jnp.int32)
    num_accepted = accept + 1
    recovered_counts = 1 - accept
    return (output_token_ids, num_accepted, accept, recovered_counts, accept)


# speedup vs baseline: 1.8420x; 1.4243x over previous
import jax
import jax.numpy as jnp
from jax import lax
from jax.experimental import pallas as pl
from jax.experimental.pallas import tpu as pltpu
from jax.experimental.pallas import tpu_sc as plsc

B = 128
V = 100000
CH = 20000
NCH = V // CH
SUB = 800
GPS = SUB // 16
SPC = CH // SUB
SPR = V // SUB
SPAD = 128

_info = plsc.get_sparse_core_info()
NC, NS = _info.num_cores, _info.num_subcores
NW = NC * NS
RPW = B // NW

_BIG = 3.0e38


def _iota16():
    return lax.iota(jnp.int32, 16)


def _get1(ref, idx):
    return plsc.load_gather(ref, [jnp.full((16,), idx, jnp.int32)])[0]


def _set1(ref, idx, val):
    plsc.store_scatter(ref, [jnp.full((16,), idx, jnp.int32)],
                       jnp.full((16,), val), mask=_iota16() == 0)


def _sc_body(d_hbm, t_hbm, ids_hbm, bon_hbm, u_hbm, ru_hbm, out_hbm,
             dbufs, tbufs, d3, t3, dq, tq, ids_v, bon_v, u_v, ru_v,
             subsums, outbuf, sems):
    wid = lax.axis_index("s") * NC + lax.axis_index("c")

    pltpu.sync_copy(ids_hbm, ids_v)
    pltpu.sync_copy(bon_hbm, bon_v)
    pltpu.sync_copy(u_hbm, u_v)
    pltpu.sync_copy(ru_hbm, ru_v)

    rows = [wid * RPW + i for i in range(RPW)]
    steps = [(i, c) for i in range(RPW) for c in range(NCH)]

    def start(k):
        i, c = steps[k]
        b = k % 2
        pltpu.async_copy(d_hbm.at[rows[i], pl.ds(c * CH, CH)], dbufs[b],
                         sems[b])
        pltpu.async_copy(t_hbm.at[rows[i], pl.ds(c * CH, CH)], tbufs[b],
                         sems[b])

    def wait(k):
        b = k % 2
        pltpu.make_async_copy(d_hbm.at[rows[0], pl.ds(0, CH)], dbufs[b],
                              sems[b]).wait()
        pltpu.make_async_copy(t_hbm.at[rows[0], pl.ds(0, CH)], tbufs[b],
                              sems[b]).wait()

    start(0)
    outacc = jnp.zeros((16,), jnp.int32)
    running = jnp.float32(0.0)
    for k, (i, c) in enumerate(steps):
        r = rows[i]
        b = k % 2
        if k + 1 < len(steps):
            start(k + 1)
        wait(k)
        dbuf, tbuf = dbufs[b], tbufs[b]

        def sub_body(s, run2, dbuf=dbuf, tbuf=tbuf, c=c):
            o0 = s * SUB
            accs = [jnp.zeros((16,), jnp.float32) for _ in range(4)]
            for g in range(GPS):
                o = o0 + g * 16
                dd = dbuf[pl.ds(o, 16)]
                tt = tbuf[pl.ds(o, 16)]
                accs[g % 4] = accs[g % 4] + jnp.maximum(tt - dd,
                                                       jnp.float32(0.0))
            acc = (accs[0] + accs[1]) + (accs[2] + accs[3])
            run2 = run2 + jnp.sum(acc)
            _set1(subsums, c * SPC + s, run2)
            return run2

        running = lax.fori_loop(0, SPC, sub_body, running)

        if c == NCH - 1:
            total = running
            running = jnp.float32(0.0)

            tid = _get1(ids_v, r)
            base8 = jnp.minimum((tid // 8) * 8, V - 16)
            pltpu.sync_copy(d_hbm.at[r, pl.ds(base8, 16)], dq)
            pltpu.sync_copy(t_hbm.at[r, pl.ds(base8, 16)], tq)
            off = tid - base8
            q = _get1(dq, off)
            p = _get1(tq, off)
            accept = (_get1(u_v, r) * q) < p

            plsc.store_scatter(subsums, [_iota16() + SPR],
                               jnp.full((16,), jnp.float32(_BIG)),
                               mask=_iota16() < (SPAD - SPR))
            thresh = _get1(ru_v, r) * total

            def p2(j, cnt):
                v = subsums[pl.ds(j * 16, 16)]
                return cnt + plsc.all_reduce_population_count(v < thresh)

            cntv = lax.fori_loop(0, SPAD // 16, p2,
                                 jnp.zeros((16,), jnp.int32))
            gidx = jnp.minimum(jnp.max(cntv), SPR - 1)
            base = jnp.where(gidx > 0,
                             _get1(subsums, jnp.maximum(gidx - 1, 0)),
                             jnp.float32(0.0))

            pltpu.sync_copy(d_hbm.at[r, pl.ds(gidx * SUB, SUB)], d3)
            pltpu.sync_copy(t_hbm.at[r, pl.ds(gidx * SUB, SUB)], t3)

            def p3(g, carry):
                bb, cv = carry
                dd = d3[pl.ds(g * 16, 16)]
                tt = t3[pl.ds(g * 16, 16)]
                res = jnp.maximum(tt - dd, jnp.float32(0.0))
                cum = plsc.cumsum(res)
                cv = cv + plsc.all_reduce_population_count(
                    (bb + cum) < thresh)
                return bb + jnp.sum(res), cv

            _, cntv3 = lax.fori_loop(0, GPS, p3,
                                     (base, jnp.zeros((16,), jnp.int32)))
            recovered = jnp.minimum(gidx * SUB + jnp.max(cntv3), V - 1)

            out0 = jnp.where(accept, tid, recovered)
            out1 = jnp.where(accept, _get1(bon_v, r), jnp.int32(-1))
            outacc = jnp.where(_iota16() == 2 * i, jnp.full((16,), out0),
                               outacc)
            outacc = jnp.where(_iota16() == 2 * i + 1,
                               jnp.full((16,), out1), outacc)

    outbuf[...] = outacc
    pltpu.sync_copy(outbuf.at[pl.ds(0, 2 * RPW)],
                    out_hbm.at[pl.ds(wid * 2 * RPW, 2 * RPW)])


@jax.jit
def _sc_sampler(draft_probs, target_probs, draft_token_ids,
                bonus_token_ids, uniform_samples, residual_uniform):
    mesh = plsc.VectorSubcoreMesh(core_axis_name="c", subcore_axis_name="s")
    return pl.kernel(
        _sc_body,
        out_type=jax.ShapeDtypeStruct((B * 2,), jnp.int32),
        mesh=mesh,
        compiler_params=pltpu.CompilerParams(use_tc_tiling_on_sc=False,
                                             needs_layout_passes=False),
        scratch_types=[
            [pltpu.VMEM((CH,), jnp.float32) for _ in range(2)],
            [pltpu.VMEM((CH,), jnp.float32) for _ in range(2)],
            pltpu.VMEM((SUB,), jnp.float32),
            pltpu.VMEM((SUB,), jnp.float32),
            pltpu.VMEM((16,), jnp.float32),
            pltpu.VMEM((16,), jnp.float32),
            pltpu.VMEM((B,), jnp.int32),
            pltpu.VMEM((B,), jnp.int32),
            pltpu.VMEM((B,), jnp.float32),
            pltpu.VMEM((B,), jnp.float32),
            pltpu.VMEM((SPAD,), jnp.float32),
            pltpu.VMEM((16,), jnp.int32),
            [pltpu.SemaphoreType.DMA for _ in range(2)],
        ],
    )(draft_probs, target_probs, draft_token_ids, bonus_token_ids,
      uniform_samples, residual_uniform)


def kernel(draft_probs, target_probs, draft_token_ids, bonus_token_ids,
           num_draft_tokens, uniform_samples, residual_uniform):
    del num_draft_tokens
    flat = _sc_sampler(draft_probs, target_probs, draft_token_ids,
                       bonus_token_ids, uniform_samples, residual_uniform)
    output_token_ids = flat.reshape(B, 2)
    accept = (output_token_ids[:, 1] != jnp.int32(-1)).astype(jnp.int32)
    num_accepted = accept + 1
    recovered_counts = 1 - accept
    return (output_token_ids, num_accepted, accept, recovered_counts, accept)


# speedup vs baseline: 2.6333x; 1.4296x over previous
import jax
import jax.numpy as jnp
from jax import lax
from jax.experimental import pallas as pl
from jax.experimental.pallas import tpu as pltpu
from jax.experimental.pallas import tpu_sc as plsc

B = 128
V = 100000
HW = 50048
CW = 2560
NFULL = 19
SUBW = 640
SPH = 79
GP3 = SUBW // 16

_info = plsc.get_sparse_core_info()
NC, NS = _info.num_cores, _info.num_subcores


def _iota16():
    return lax.iota(jnp.int32, 16)


def _get1(ref, idx):
    return plsc.load_gather(ref, [jnp.full((16,), idx, jnp.int32)])[0]


def _get2(ref, i, j):
    return plsc.load_gather(ref, [jnp.full((16,), i, jnp.int32),
                                  jnp.full((16,), j, jnp.int32)])[0]


def _set2(ref, i, j, val):
    plsc.store_scatter(ref, [jnp.full((16,), i, jnp.int32),
                             jnp.full((16,), j, jnp.int32)],
                       jnp.full((16,), val), mask=_iota16() == 0)


def _sc_body(d_hbm, t_hbm, ids_hbm, bon_hbm, u_hbm, ru_hbm, out_hbm,
             dblks, tblks, dtl, ttl, dtl32, ttl32, qd32, qt32,
             subsums, cumbuf,
             ids_v, bon_v, u_v, ru_v, xf, xi, sharedF, sharedI,
             outb, sems):
    cid = lax.axis_index("c")
    sid = lax.axis_index("s")
    gt = 8 * cid + sid // 2
    h = sid % 2
    r0 = pl.multiple_of(8 * gt, 8)
    hoff = HW * h
    iota = _iota16()

    pltpu.sync_copy(ids_hbm, ids_v)
    pltpu.sync_copy(bon_hbm, bon_v)
    pltpu.sync_copy(u_hbm, u_v)
    pltpu.sync_copy(ru_hbm, ru_v)

    chunks = [(CW * k, CW, 4) for k in range(NFULL)] + [(CW * NFULL, 1280, 2)]

    def startch(k):
        off, w, _ = chunks[k]
        b = k % 2
        col = pl.multiple_of(hoff + off, 128)
        pltpu.async_copy(d_hbm.at[pl.ds(r0, 8), pl.ds(col, w)],
                         dblks[b].at[:, pl.ds(0, w)], sems[b])
        pltpu.async_copy(t_hbm.at[pl.ds(r0, 8), pl.ds(col, w)],
                         tblks[b].at[:, pl.ds(0, w)], sems[b])

    def waitch(k):
        off, w, _ = chunks[k]
        b = k % 2
        col = pl.multiple_of(hoff + off, 128)
        pltpu.make_async_copy(d_hbm.at[pl.ds(r0, 8), pl.ds(col, w)],
                              dblks[b].at[:, pl.ds(0, w)], sems[b]).wait()
        pltpu.make_async_copy(t_hbm.at[pl.ds(r0, 8), pl.ds(col, w)],
                              tblks[b].at[:, pl.ds(0, w)], sems[b]).wait()

    for i in range(8):
        _set2(subsums, i, SPH, jnp.float32(0.0))

    startch(0)
    for k, (off, w, nsub) in enumerate(chunks):
        if k + 1 < len(chunks):
            startch(k + 1)
        waitch(k)
        db, tb = dblks[k % 2], tblks[k % 2]

        def row_body(row, _, db=db, tb=tb, k=k):
            rowv = jnp.full((16,), row, jnp.int32)

            def sub_body(s, _2):
                sbase = s * SUBW

                def grp(g, accs):
                    out = []
                    for u in range(4):
                        colv = sbase + g * 64 + u * 16 + iota
                        dd = plsc.load_gather(db, [rowv, colv])
                        tt = plsc.load_gather(tb, [rowv, colv])
                        out.append(accs[u] + jnp.maximum(
                            tt - dd, jnp.float32(0.0)))
                    return tuple(out)

                z = jnp.zeros((16,), jnp.float32)
                a = lax.fori_loop(0, GP3 // 4, grp, (z, z, z, z))
                tot = jnp.sum((a[0] + a[1]) + (a[2] + a[3]))
                plsc.store_scatter(
                    subsums,
                    [rowv, jnp.full((16,), 4 * k, jnp.int32) + s],
                    jnp.full((16,), tot), mask=iota == 0)
                return _2

            return lax.fori_loop(0, nsub, sub_body, _)

        lax.fori_loop(0, 8, row_body, jnp.int32(0))

    @pl.when(h == 0)
    def _():
        pltpu.sync_copy(d_hbm.at[pl.ds(r0, 8), pl.ds(49920, 128)], dtl)
        pltpu.sync_copy(t_hbm.at[pl.ds(r0, 8), pl.ds(49920, 128)], ttl)

    @pl.when(h == 1)
    def _():
        pltpu.sync_copy(d_hbm.at[pl.ds(r0, 8), pl.ds(99968, 32)], dtl32)
        pltpu.sync_copy(t_hbm.at[pl.ds(r0, 8), pl.ds(99968, 32)], ttl32)

    def _tail_accum(dref, tref, ngrp):
        def tail_body(row, _):
            rowv = jnp.full((16,), row, jnp.int32)

            def tgrp(g, acc):
                colv = g * 16 + iota
                dd = plsc.load_gather(dref, [rowv, colv])
                tt = plsc.load_gather(tref, [rowv, colv])
                return acc + jnp.maximum(tt - dd, jnp.float32(0.0))

            acc = lax.fori_loop(0, ngrp, tgrp,
                                jnp.zeros((16,), jnp.float32))
            plsc.store_scatter(subsums,
                               [rowv, jnp.full((16,), 78, jnp.int32)],
                               jnp.full((16,), jnp.sum(acc)),
                               mask=iota == 0)
            return _

        lax.fori_loop(0, 8, tail_body, jnp.int32(0))

    @pl.when(h == 0)
    def _():
        _tail_accum(dtl, ttl, 8)

    @pl.when(h == 1)
    def _():
        _tail_accum(dtl32, ttl32, 2)

    mytotv = jnp.zeros((16,), jnp.float32)
    for i in range(8):
        acc = jnp.zeros((16,), jnp.float32)

        def p2a(j, acc, i=i):
            return acc + subsums[i, pl.ds(j * 16, 16)]

        acc = lax.fori_loop(0, 5, p2a, acc)
        mytotv = jnp.where(iota == i, jnp.full((16,), jnp.sum(acc)), mytotv)

    slot = pl.multiple_of(sid * 16, 8)
    pslot = pl.multiple_of((sid ^ 1) * 16, 8)
    xf[...] = mytotv
    pltpu.sync_copy(xf, sharedF.at[pl.ds(slot, 16)])
    plsc.subcore_barrier()
    pltpu.sync_copy(sharedF.at[pl.ds(pslot, 16)], xf)
    ptotv = xf[...]

    hv = jnp.full((16,), h, jnp.int32)
    T0v = jnp.where(hv == 0, mytotv, ptotv)
    T1v = jnp.where(hv == 0, ptotv, mytotv)
    ridx = jnp.minimum(r0 + iota, B - 1)
    ruv = plsc.load_gather(ru_v, [ridx])
    threshv = ruv * (T0v + T1v)
    c0v = T0v >= threshv
    bstartv = jnp.where(hv == 0, jnp.zeros((16,), jnp.float32), T0v)

    gidxs, bases = [], []
    for i in range(8):
        t_r = threshv[i]
        b0 = bstartv[i]

        def p2b(j, carry, i=i, t_r=t_r):
            bb, cnt = carry
            v = subsums[i, pl.ds(j * 16, 16)]
            cum = bb + plsc.cumsum(v)
            cumbuf[i, pl.ds(j * 16, 16)] = cum
            cnt = cnt + plsc.all_reduce_population_count(cum < t_r)
            return bb + jnp.sum(v), cnt

        _, cntv = lax.fori_loop(0, 5, p2b,
                                (b0, jnp.zeros((16,), jnp.int32)))
        gidx = jnp.minimum(jnp.max(cntv), SPH - 1)
        base = jnp.where(gidx > 0,
                         _get2(cumbuf, i, jnp.maximum(gidx - 1, 0)), b0)
        gidxs.append(gidx)
        bases.append(base)

    descs = []
    for i in range(8):
        offrel = pl.multiple_of(
            jnp.minimum(SUBW * gidxs[i], 49280), 128)
        off = pl.multiple_of(hoff + offrel, 128)
        bi, seg = i // 4, SUBW * (i % 4)
        for src, dstb in ((d_hbm, dblks[bi]), (t_hbm, tblks[bi])):
            c = pltpu.async_copy(src.at[pl.ds(r0, 8), pl.ds(off, SUBW)],
                                 dstb.at[:, pl.ds(seg, SUBW)], sems[0])
            descs.append(c)
    for c in descs:
        c.wait()

    recv = jnp.zeros((16,), jnp.int32)
    for i in range(8):
        bi, seg = i // 4, SUBW * (i % 4)
        t_r = threshv[i]
        rowv = jnp.full((16,), i, jnp.int32)
        zi = jnp.zeros((16,), jnp.int32)

        def p3_main(bi=bi, seg=seg, t_r=t_r, rowv=rowv, base=bases[i]):
            def stepm(g, carry):
                bb, cv = carry
                colv = seg + g * 16 + iota
                dd = plsc.load_gather(dblks[bi], [rowv, colv])
                tt = plsc.load_gather(tblks[bi], [rowv, colv])
                res = jnp.maximum(tt - dd, jnp.float32(0.0))
                cum = plsc.cumsum(res)
                cv = cv + plsc.all_reduce_population_count(
                    (bb + cum) < t_r)
                return bb + jnp.sum(res), cv

            return lax.fori_loop(0, GP3, stepm, (base, zi))[1]

        def p3_tail(dref, tref, ngrp, t_r=t_r, rowv=rowv, base=bases[i]):
            def stept(g, carry):
                bb, cv = carry
                colv = g * 16 + iota
                dd = plsc.load_gather(dref, [rowv, colv])
                tt = plsc.load_gather(tref, [rowv, colv])
                res = jnp.maximum(tt - dd, jnp.float32(0.0))
                cum = plsc.cumsum(res)
                cv = cv + plsc.all_reduce_population_count(
                    (bb + cum) < t_r)
                return bb + jnp.sum(res), cv

            return lax.fori_loop(0, ngrp, stept, (base, zi))[1]

        cv = lax.cond(
            gidxs[i] == SPH - 1,
            lambda: lax.cond(h == 1,
                             lambda: p3_tail(dtl32, ttl32, 2),
                             lambda: p3_tail(dtl, ttl, 8)),
            p3_main)
        rec = jnp.minimum(hoff + SUBW * gidxs[i] + jnp.max(cv), V - 1)
        recv = jnp.where(iota == i, jnp.full((16,), rec), recv)

    xi[...] = recv
    pltpu.sync_copy(xi, sharedI.at[pl.ds(slot, 16)])
    plsc.subcore_barrier()

    @pl.when(h == 0)
    def _():
        pltpu.sync_copy(sharedI.at[pl.ds(pslot, 16)], xi)
        prec = xi[...]
        rec_final = jnp.where(c0v, recv, prec)

        outv = jnp.zeros((16,), jnp.int32)
        for i in range(8):
            tid_i = _get1(ids_v, r0 + i)
            intail = tid_i >= 99968
            walign = pl.multiple_of((tid_i // 128) * 128, 128)
            seg = 128 * i

            @pl.when(jnp.logical_not(intail))
            def _(walign=walign, seg=seg):
                pltpu.sync_copy(
                    d_hbm.at[pl.ds(r0, 8), pl.ds(walign, 128)],
                    dblks[1].at[:, pl.ds(seg, 128)])
                pltpu.sync_copy(
                    t_hbm.at[pl.ds(r0, 8), pl.ds(walign, 128)],
                    tblks[1].at[:, pl.ds(seg, 128)])

            @pl.when(intail)
            def _():
                pltpu.sync_copy(
                    d_hbm.at[pl.ds(r0, 8), pl.ds(99968, 32)], qd32)
                pltpu.sync_copy(
                    t_hbm.at[pl.ds(r0, 8), pl.ds(99968, 32)], qt32)

            rowv = jnp.full((16,), i, jnp.int32)

            def qp_main(rowv=rowv, seg=seg, tid_i=tid_i, walign=walign):
                locv = jnp.full((16,), seg + (tid_i - walign), jnp.int32)
                return (plsc.load_gather(dblks[1], [rowv, locv])[0],
                        plsc.load_gather(tblks[1], [rowv, locv])[0])

            def qp_tail(rowv=rowv, tid_i=tid_i):
                locv = jnp.full((16,), tid_i - 99968, jnp.int32)
                return (plsc.load_gather(qd32, [rowv, locv])[0],
                        plsc.load_gather(qt32, [rowv, locv])[0])

            q, p = lax.cond(intail, qp_tail, qp_main)
            accept = (_get1(u_v, r0 + i) * q) < p
            out0 = jnp.where(accept, tid_i, rec_final[i])
            out1 = jnp.where(accept, _get1(bon_v, r0 + i), jnp.int32(-1))
            outv = jnp.where(iota == 2 * i, jnp.full((16,), out0), outv)
            outv = jnp.where(iota == 2 * i + 1, jnp.full((16,), out1), outv)

        outb[...] = outv
        pltpu.sync_copy(outb, out_hbm.at[pl.ds(pl.multiple_of(gt * 16, 8),
                                               16)])


@jax.jit
def _sc_sampler(draft_probs, target_probs, draft_token_ids,
                bonus_token_ids, uniform_samples, residual_uniform):
    mesh = plsc.VectorSubcoreMesh(core_axis_name="c", subcore_axis_name="s")
    return pl.kernel(
        _sc_body,
        out_type=jax.ShapeDtypeStruct((B * 2,), jnp.int32),
        mesh=mesh,
        compiler_params=pltpu.CompilerParams(use_tc_tiling_on_sc=True,
                                             needs_layout_passes=False),
        scratch_types=[
            [pltpu.VMEM((8, CW), jnp.float32) for _ in range(2)],
            [pltpu.VMEM((8, CW), jnp.float32) for _ in range(2)],
            pltpu.VMEM((8, 128), jnp.float32),
            pltpu.VMEM((8, 128), jnp.float32),
            pltpu.VMEM((8, 32), jnp.float32),
            pltpu.VMEM((8, 32), jnp.float32),
            pltpu.VMEM((8, 32), jnp.float32),
            pltpu.VMEM((8, 32), jnp.float32),
            pltpu.VMEM((8, 128), jnp.float32),
            pltpu.VMEM((8, 128), jnp.float32),
            pltpu.VMEM((B,), jnp.int32),
            pltpu.VMEM((B,), jnp.int32),
            pltpu.VMEM((B,), jnp.float32),
            pltpu.VMEM((B,), jnp.float32),
            pltpu.VMEM((16,), jnp.float32),
            pltpu.VMEM((16,), jnp.int32),
            pltpu.VMEM_SHARED((256,), jnp.float32),
            pltpu.VMEM_SHARED((256,), jnp.int32),
            pltpu.VMEM((16,), jnp.int32),
            [pltpu.SemaphoreType.DMA for _ in range(2)],
        ],
    )(draft_probs, target_probs, draft_token_ids, bonus_token_ids,
      uniform_samples, residual_uniform)


def kernel(draft_probs, target_probs, draft_token_ids, bonus_token_ids,
           num_draft_tokens, uniform_samples, residual_uniform):
    del num_draft_tokens
    flat = _sc_sampler(draft_probs, target_probs, draft_token_ids,
                       bonus_token_ids, uniform_samples, residual_uniform)
    output_token_ids = flat.reshape(B, 2)
    accept = (output_token_ids[:, 1] != jnp.int32(-1)).astype(jnp.int32)
    num_accepted = accept + 1
    recovered_counts = 1 - accept
    return (output_token_ids, num_accepted, accept, recovered_counts, accept)


# speedup vs baseline: 2.7426x; 1.0415x over previous
import jax
import jax.numpy as jnp
from jax import lax
from jax.experimental import pallas as pl
from jax.experimental.pallas import tpu as pltpu
from jax.experimental.pallas import tpu_sc as plsc

B = 128
V = 100000
HW = 50048
CW = 2560
NFULL = 19
SUBW = 1280
SPH = 40
SPAD = 48
GP3 = SUBW // 16

_info = plsc.get_sparse_core_info()
NC, NS = _info.num_cores, _info.num_subcores


def _iota16():
    return lax.iota(jnp.int32, 16)


def _get1(ref, idx):
    return plsc.load_gather(ref, [jnp.full((16,), idx, jnp.int32)])[0]


def _get2(ref, i, j):
    return plsc.load_gather(ref, [jnp.full((16,), i, jnp.int32),
                                  jnp.full((16,), j, jnp.int32)])[0]


def _set2(ref, i, j, val):
    plsc.store_scatter(ref, [jnp.full((16,), i, jnp.int32),
                             jnp.full((16,), j, jnp.int32)],
                       jnp.full((16,), val), mask=_iota16() == 0)


def _sc_body(d_hbm, t_hbm, ids_hbm, bon_hbm, u_hbm, ru_hbm, out_hbm,
             dblks, tblks, dtl, ttl, dtl32, ttl32, qd32, qt32,
             subsums, cumbuf,
             ids_v, bon_v, u_v, ru_v, xf, xi, sharedF, sharedI,
             outb, sems):
    cid = lax.axis_index("c")
    sid = lax.axis_index("s")
    gt = 8 * cid + sid // 2
    h = sid % 2
    r0 = pl.multiple_of(8 * gt, 8)
    hoff = HW * h
    iota = _iota16()

    pltpu.sync_copy(ru_hbm, ru_v)

    chunks = [(CW * k, CW, 2) for k in range(NFULL)] + [(CW * NFULL, 1280, 1)]

    def startch(k):
        off, w, _ = chunks[k]
        b = k % 2
        col = pl.multiple_of(hoff + off, 128)
        pltpu.async_copy(d_hbm.at[pl.ds(r0, 8), pl.ds(col, w)],
                         dblks[b].at[:, pl.ds(0, w)], sems[b])
        pltpu.async_copy(t_hbm.at[pl.ds(r0, 8), pl.ds(col, w)],
                         tblks[b].at[:, pl.ds(0, w)], sems[b])

    def waitch(k):
        off, w, _ = chunks[k]
        b = k % 2
        col = pl.multiple_of(hoff + off, 128)
        pltpu.make_async_copy(d_hbm.at[pl.ds(r0, 8), pl.ds(col, w)],
                              dblks[b].at[:, pl.ds(0, w)], sems[b]).wait()
        pltpu.make_async_copy(t_hbm.at[pl.ds(r0, 8), pl.ds(col, w)],
                              tblks[b].at[:, pl.ds(0, w)], sems[b]).wait()

    for i in range(8):
        plsc.store_scatter(subsums,
                           [jnp.full((16,), i, jnp.int32), SPH + iota],
                           jnp.zeros((16,), jnp.float32),
                           mask=iota < (SPAD - SPH))

    startch(0)
    for k, (off, w, nsub) in enumerate(chunks):
        if k + 1 < len(chunks):
            startch(k + 1)
        waitch(k)
        db, tb = dblks[k % 2], tblks[k % 2]

        def row_body(row, _, db=db, tb=tb, k=k):
            rowv = jnp.full((16,), row, jnp.int32)

            def sub_body(s, _2):
                def grp(g, carry):
                    colv = carry[0]
                    accs = list(carry[1:])
                    for u in range(8):
                        cu = colv + u * 16
                        dd = plsc.load_gather(db, [rowv, cu])
                        tt = plsc.load_gather(tb, [rowv, cu])
                        accs[u % 4] = accs[u % 4] + jnp.maximum(
                            tt - dd, jnp.float32(0.0))
                    return (colv + 128,) + tuple(accs)

                z = jnp.zeros((16,), jnp.float32)
                col0 = s * SUBW + iota
                a = lax.fori_loop(0, GP3 // 8, grp, (col0, z, z, z, z))
                tot = jnp.sum((a[1] + a[2]) + (a[3] + a[4]))
                plsc.store_scatter(
                    subsums,
                    [rowv, jnp.full((16,), 2 * k, jnp.int32) + s],
                    jnp.full((16,), tot), mask=iota == 0)
                return _2

            return lax.fori_loop(0, nsub, sub_body, _)

        lax.fori_loop(0, 8, row_body, jnp.int32(0))

    @pl.when(h == 0)
    def _():
        pltpu.sync_copy(d_hbm.at[pl.ds(r0, 8), pl.ds(49920, 128)], dtl)
        pltpu.sync_copy(t_hbm.at[pl.ds(r0, 8), pl.ds(49920, 128)], ttl)

    @pl.when(h == 1)
    def _():
        pltpu.sync_copy(d_hbm.at[pl.ds(r0, 8), pl.ds(99968, 32)], dtl32)
        pltpu.sync_copy(t_hbm.at[pl.ds(r0, 8), pl.ds(99968, 32)], ttl32)

    def _tail_accum(dref, tref, ngrp):
        def tail_body(row, _):
            rowv = jnp.full((16,), row, jnp.int32)

            def tgrp(g, acc):
                colv = g * 16 + iota
                dd = plsc.load_gather(dref, [rowv, colv])
                tt = plsc.load_gather(tref, [rowv, colv])
                return acc + jnp.maximum(tt - dd, jnp.float32(0.0))

            acc = lax.fori_loop(0, ngrp, tgrp,
                                jnp.zeros((16,), jnp.float32))
            plsc.store_scatter(subsums,
                               [rowv, jnp.full((16,), SPH - 1, jnp.int32)],
                               jnp.full((16,), jnp.sum(acc)),
                               mask=iota == 0)
            return _

        lax.fori_loop(0, 8, tail_body, jnp.int32(0))

    @pl.when(h == 0)
    def _():
        _tail_accum(dtl, ttl, 8)

    @pl.when(h == 1)
    def _():
        _tail_accum(dtl32, ttl32, 2)

    mytotv = jnp.zeros((16,), jnp.float32)
    for i in range(8):
        acc = jnp.zeros((16,), jnp.float32)

        def p2a(j, acc, i=i):
            return acc + subsums[i, pl.ds(j * 16, 16)]

        acc = lax.fori_loop(0, SPAD // 16, p2a, acc)
        mytotv = jnp.where(iota == i, jnp.full((16,), jnp.sum(acc)), mytotv)

    slot = pl.multiple_of(sid * 16, 8)
    pslot = pl.multiple_of((sid ^ 1) * 16, 8)
    xf[...] = mytotv
    pltpu.sync_copy(xf, sharedF.at[pl.ds(slot, 16)])
    plsc.subcore_barrier()
    pltpu.sync_copy(sharedF.at[pl.ds(pslot, 16)], xf)
    ptotv = xf[...]

    hv = jnp.full((16,), h, jnp.int32)
    T0v = jnp.where(hv == 0, mytotv, ptotv)
    T1v = jnp.where(hv == 0, ptotv, mytotv)
    ridx = jnp.minimum(r0 + iota, B - 1)
    ruv = plsc.load_gather(ru_v, [ridx])
    threshv = ruv * (T0v + T1v)
    c0v = T0v >= threshv
    bstartv = jnp.where(hv == 0, jnp.zeros((16,), jnp.float32), T0v)

    gidxs, bases = [], []
    for i in range(8):
        t_r = threshv[i]
        b0 = bstartv[i]

        def p2b(j, carry, i=i, t_r=t_r):
            bb, cnt = carry
            v = subsums[i, pl.ds(j * 16, 16)]
            cum = bb + plsc.cumsum(v)
            cumbuf[i, pl.ds(j * 16, 16)] = cum
            cnt = cnt + plsc.all_reduce_population_count(cum < t_r)
            return bb + jnp.sum(v), cnt

        _, cntv = lax.fori_loop(0, SPAD // 16, p2b,
                                (b0, jnp.zeros((16,), jnp.int32)))
        gidx = jnp.minimum(jnp.max(cntv), SPH - 1)
        base = jnp.where(gidx > 0,
                         _get2(cumbuf, i, jnp.maximum(gidx - 1, 0)), b0)
        gidxs.append(gidx)
        bases.append(base)

    recv = jnp.zeros((16,), jnp.int32)
    for wave in range(2):
        descs = []
        for j in range(4):
            i = 4 * wave + j
            offrel = pl.multiple_of(
                jnp.minimum(SUBW * gidxs[i], NFULL * CW), 128)
            off = pl.multiple_of(hoff + offrel, 128)
            bi, seg = j // 2, SUBW * (j % 2)
            for src, dstb in ((d_hbm, dblks[bi]), (t_hbm, tblks[bi])):
                c = pltpu.async_copy(
                    src.at[pl.ds(r0, 8), pl.ds(off, SUBW)],
                    dstb.at[:, pl.ds(seg, SUBW)], sems[0])
                descs.append(c)
        for c in descs:
            c.wait()

        for j in range(4):
            i = 4 * wave + j
            bi, seg = j // 2, SUBW * (j % 2)
            t_r = threshv[i]
            rowv = jnp.full((16,), i, jnp.int32)
            zi = jnp.zeros((16,), jnp.int32)

            def p3_main(bi=bi, seg=seg, t_r=t_r, rowv=rowv, base=bases[i]):
                def stepm(g, carry):
                    bb, cv = carry
                    colv = seg + g * 16 + iota
                    dd = plsc.load_gather(dblks[bi], [rowv, colv])
                    tt = plsc.load_gather(tblks[bi], [rowv, colv])
                    res = jnp.maximum(tt - dd, jnp.float32(0.0))
                    cum = plsc.cumsum(res)
                    cv = cv + plsc.all_reduce_population_count(
                        (bb + cum) < t_r)
                    return bb + jnp.sum(res), cv

                return lax.fori_loop(0, GP3, stepm, (base, zi))[1]

            def p3_tail(dref, tref, ngrp, t_r=t_r, rowv=rowv,
                        base=bases[i]):
                def stept(g, carry):
                    bb, cv = carry
                    colv = g * 16 + iota
                    dd = plsc.load_gather(dref, [rowv, colv])
                    tt = plsc.load_gather(tref, [rowv, colv])
                    res = jnp.maximum(tt - dd, jnp.float32(0.0))
                    cum = plsc.cumsum(res)
                    cv = cv + plsc.all_reduce_population_count(
                        (bb + cum) < t_r)
                    return bb + jnp.sum(res), cv

                return lax.fori_loop(0, ngrp, stept, (base, zi))[1]

            cv = lax.cond(
                gidxs[i] == SPH - 1,
                lambda: lax.cond(h == 1,
                                 lambda: p3_tail(dtl32, ttl32, 2),
                                 lambda: p3_tail(dtl, ttl, 8)),
                p3_main)
            rec = jnp.minimum(hoff + SUBW * gidxs[i] + jnp.max(cv), V - 1)
            recv = jnp.where(iota == i, jnp.full((16,), rec), recv)

    xi[...] = recv
    pltpu.sync_copy(xi, sharedI.at[pl.ds(slot, 16)])
    plsc.subcore_barrier()

    @pl.when(h == 0)
    def _():
        pltpu.sync_copy(ids_hbm, ids_v)
        pltpu.sync_copy(bon_hbm, bon_v)
        pltpu.sync_copy(u_hbm, u_v)
        pltpu.sync_copy(sharedI.at[pl.ds(pslot, 16)], xi)
        prec = xi[...]
        rec_final = jnp.where(c0v, recv, prec)

        outv = jnp.zeros((16,), jnp.int32)
        for i in range(8):
            tid_i = _get1(ids_v, r0 + i)
            intail = tid_i >= 99968
            walign = pl.multiple_of((tid_i // 128) * 128, 128)
            seg = 128 * i

            @pl.when(jnp.logical_not(intail))
            def _(walign=walign, seg=seg):
                pltpu.sync_copy(
                    d_hbm.at[pl.ds(r0, 8), pl.ds(walign, 128)],
                    dblks[1].at[:, pl.ds(seg, 128)])
                pltpu.sync_copy(
                    t_hbm.at[pl.ds(r0, 8), pl.ds(walign, 128)],
                    tblks[1].at[:, pl.ds(seg, 128)])

            @pl.when(intail)
            def _():
                pltpu.sync_copy(
                    d_hbm.at[pl.ds(r0, 8), pl.ds(99968, 32)], qd32)
                pltpu.sync_copy(
                    t_hbm.at[pl.ds(r0, 8), pl.ds(99968, 32)], qt32)

            rowv = jnp.full((16,), i, jnp.int32)

            def qp_main(rowv=rowv, seg=seg, tid_i=tid_i, walign=walign):
                locv = jnp.full((16,), seg + (tid_i - walign), jnp.int32)
                return (plsc.load_gather(dblks[1], [rowv, locv])[0],
                        plsc.load_gather(tblks[1], [rowv, locv])[0])

            def qp_tail(rowv=rowv, tid_i=tid_i):
                locv = jnp.full((16,), tid_i - 99968, jnp.int32)
                return (plsc.load_gather(qd32, [rowv, locv])[0],
                        plsc.load_gather(qt32, [rowv, locv])[0])

            q, p = lax.cond(intail, qp_tail, qp_main)
            accept = (_get1(u_v, r0 + i) * q) < p
            out0 = jnp.where(accept, tid_i, rec_final[i])
            out1 = jnp.where(accept, _get1(bon_v, r0 + i), jnp.int32(-1))
            outv = jnp.where(iota == 2 * i, jnp.full((16,), out0), outv)
            outv = jnp.where(iota == 2 * i + 1, jnp.full((16,), out1), outv)

        outb[...] = outv
        pltpu.sync_copy(outb, out_hbm.at[pl.ds(pl.multiple_of(gt * 16, 8),
                                               16)])


@jax.jit
def _sc_sampler(draft_probs, target_probs, draft_token_ids,
                bonus_token_ids, uniform_samples, residual_uniform):
    mesh = plsc.VectorSubcoreMesh(core_axis_name="c", subcore_axis_name="s")
    return pl.kernel(
        _sc_body,
        out_type=jax.ShapeDtypeStruct((B * 2,), jnp.int32),
        mesh=mesh,
        compiler_params=pltpu.CompilerParams(use_tc_tiling_on_sc=True,
                                             needs_layout_passes=False),
        scratch_types=[
            [pltpu.VMEM((8, CW), jnp.float32) for _ in range(2)],
            [pltpu.VMEM((8, CW), jnp.float32) for _ in range(2)],
            pltpu.VMEM((8, 128), jnp.float32),
            pltpu.VMEM((8, 128), jnp.float32),
            pltpu.VMEM((8, 32), jnp.float32),
            pltpu.VMEM((8, 32), jnp.float32),
            pltpu.VMEM((8, 32), jnp.float32),
            pltpu.VMEM((8, 32), jnp.float32),
            pltpu.VMEM((8, SPAD), jnp.float32),
            pltpu.VMEM((8, SPAD), jnp.float32),
            pltpu.VMEM((B,), jnp.int32),
            pltpu.VMEM((B,), jnp.int32),
            pltpu.VMEM((B,), jnp.float32),
            pltpu.VMEM((B,), jnp.float32),
            pltpu.VMEM((16,), jnp.float32),
            pltpu.VMEM((16,), jnp.int32),
            pltpu.VMEM_SHARED((256,), jnp.float32),
            pltpu.VMEM_SHARED((256,), jnp.int32),
            pltpu.VMEM((16,), jnp.int32),
            [pltpu.SemaphoreType.DMA for _ in range(2)],
        ],
    )(draft_probs, target_probs, draft_token_ids, bonus_token_ids,
      uniform_samples, residual_uniform)


def kernel(draft_probs, target_probs, draft_token_ids, bonus_token_ids,
           num_draft_tokens, uniform_samples, residual_uniform):
    del num_draft_tokens
    flat = _sc_sampler(draft_probs, target_probs, draft_token_ids,
                       bonus_token_ids, uniform_samples, residual_uniform)
    output_token_ids = flat.reshape(B, 2)
    accept = (output_token_ids[:, 1] != jnp.int32(-1)).astype(jnp.int32)
    num_accepted = accept + 1
    recovered_counts = 1 - accept
    return (output_token_ids, num_accepted, accept, recovered_counts, accept)
